# Initial kernel scaffold; baseline (speedup 1.0000x reference)
import jax
import jax.numpy as jnp
from jax.experimental import pallas as pl


def kernel(x, route_W, route_b, noise_W, noise_b):
    raise NotImplementedError("write your pallas kernel here")


# speedup vs baseline: 1.8935x; 1.8935x over previous
import functools

import jax
import jax.numpy as jnp
from jax.experimental import pallas as pl

_N_EXPERTS = 16
_TOP_K = 2


def _router_kernel(x_ref, w_ref, b_ref, eps_ref, probs_ref, idx_ref):
    rows = x_ref.shape[0]
    acc = jnp.dot(x_ref[...], w_ref[...], preferred_element_type=jnp.float32)
    logits = acc[:, :_N_EXPERTS] + b_ref[0, :_N_EXPERTS]
    noise_raw = acc[:, _N_EXPERTS:] + b_ref[0, _N_EXPERTS:]
    noisy = logits + eps_ref[...] * jax.nn.softplus(noise_raw)

    iota = jax.lax.broadcasted_iota(jnp.int32, (rows, _N_EXPERTS), 1)
    m1 = jnp.max(noisy, axis=1, keepdims=True)
    i1 = jnp.min(jnp.where(noisy == m1, iota, _N_EXPERTS), axis=1, keepdims=True)
    masked = jnp.where(iota == i1, -jnp.inf, noisy)
    m2 = jnp.max(masked, axis=1, keepdims=True)
    i2 = jnp.min(jnp.where(masked == m2, iota, _N_EXPERTS), axis=1, keepdims=True)

    e = jnp.exp(m2 - m1)
    p1 = 1.0 / (1.0 + e)
    p2 = e / (1.0 + e)
    probs_ref[...] = jnp.where(iota == i1, p1, jnp.where(iota == i2, p2, 0.0))

    kiota = jax.lax.broadcasted_iota(jnp.int32, (rows, _TOP_K), 1)
    idx_ref[...] = jnp.where(kiota == 0, i1, i2)


@functools.partial(jax.jit, static_argnames=("block_rows",))
def _run(x, w_cat, b_cat, eps, block_rows=1024):
    n, d = x.shape
    grid = (n // block_rows,)
    return pl.pallas_call(
        _router_kernel,
        grid=grid,
        in_specs=[
            pl.BlockSpec((block_rows, d), lambda i: (i, 0)),
            pl.BlockSpec((d, 2 * _N_EXPERTS), lambda i: (0, 0)),
            pl.BlockSpec((1, 2 * _N_EXPERTS), lambda i: (0, 0)),
            pl.BlockSpec((block_rows, _N_EXPERTS), lambda i: (i, 0)),
        ],
        out_specs=[
            pl.BlockSpec((block_rows, _N_EXPERTS), lambda i: (i, 0)),
            pl.BlockSpec((block_rows, _TOP_K), lambda i: (i, 0)),
        ],
        out_shape=[
            jax.ShapeDtypeStruct((n, _N_EXPERTS), jnp.float32),
            jax.ShapeDtypeStruct((n, _TOP_K), jnp.int32),
        ],
    )(x, w_cat, b_cat, eps)


def kernel(x, route_W, route_b, noise_W, noise_b):
    n = x.shape[0]
    w_cat = jnp.concatenate([route_W, noise_W], axis=0).T
    b_cat = jnp.concatenate([route_b, noise_b], axis=0)[None, :]
    eps = jax.random.normal(jax.random.key(42), (n, _N_EXPERTS), dtype=jnp.float32)
    probs, idx = _run(x, w_cat, b_cat, eps)
    return (probs, idx)
